# Initial kernel scaffold; baseline (speedup 1.0000x reference)
import jax
import jax.numpy as jnp
from jax.experimental import pallas as pl


def kernel(q, k, v, Wq, bq, Wk, bk, Wv, bv, Wo, bo):
    raise NotImplementedError("write your pallas kernel here")


# speedup vs baseline: 6.9579x; 6.9579x over previous
import math

import numpy as np
import jax
import jax.numpy as jnp
from jax.experimental import pallas as pl

_H = 16

_HIGHEST = jax.lax.Precision.HIGHEST


def _dot(a, b):
    return jax.lax.dot_general(
        a, b, (((1,), (0,)), ((), ())),
        precision=_HIGHEST, preferred_element_type=jnp.float32)


def _dft_mats(L):
    F = L // 2 + 1
    FP = ((F + 127) // 128) * 128
    t = np.arange(L, dtype=np.int64)[:, None]
    f = np.arange(F, dtype=np.int64)[None, :]
    ang = 2.0 * np.pi * ((t * f) % L).astype(np.float64) / L
    CF = np.zeros((L, FP), np.float32)
    SF = np.zeros((L, FP), np.float32)
    CF[:, :F] = np.cos(ang)
    SF[:, :F] = np.sin(ang)
    alpha = np.full((F,), 2.0, np.float64)
    alpha[0] = 1.0
    if L % 2 == 0:
        alpha[F - 1] = 1.0
    ICc = np.zeros((FP, L), np.float32)
    ICs = np.zeros((FP, L), np.float32)
    ICc[:F, :] = (alpha[:, None] / L) * np.cos(ang.T)
    ICs[:F, :] = -(alpha[:, None] / L) * np.sin(ang.T)
    return jnp.asarray(CF), jnp.asarray(SF), jnp.asarray(ICc), jnp.asarray(ICs)


def _mm_bias_kernel(x_ref, w_ref, b_ref, o_ref):
    xb = x_ref[...].astype(jnp.bfloat16)
    wb = w_ref[...].astype(jnp.bfloat16)
    o_ref[...] = jax.lax.dot_general(
        xb, wb, (((1,), (0,)), ((), ())),
        preferred_element_type=jnp.float32) + b_ref[...]


def _matmul_bias(x, W, b, blk):
    M, K = x.shape
    N = W.shape[1]
    return pl.pallas_call(
        _mm_bias_kernel,
        grid=(M // blk,),
        in_specs=[pl.BlockSpec((blk, K), lambda i: (i, 0)),
                  pl.BlockSpec((K, N), lambda i: (0, 0)),
                  pl.BlockSpec((1, N), lambda i: (0, 0))],
        out_specs=pl.BlockSpec((blk, N), lambda i: (i, 0)),
        out_shape=jax.ShapeDtypeStruct((M, N), jnp.float32),
        interpret=False,
    )(x, W, b.reshape(1, N))


def _fwd_kernel(x_ref, cf_ref, sf_ref, xr_ref, xi_ref):
    x = x_ref[...]
    xr_ref[...] = _dot(x, cf_ref[...])
    xi_ref[...] = _dot(x, sf_ref[...])


def _fwd_fft(x, CF, SF, blk):
    M, L = x.shape
    FP = CF.shape[1]
    return pl.pallas_call(
        _fwd_kernel,
        grid=(M // blk,),
        in_specs=[pl.BlockSpec((blk, L), lambda i: (i, 0)),
                  pl.BlockSpec((L, FP), lambda i: (0, 0)),
                  pl.BlockSpec((L, FP), lambda i: (0, 0))],
        out_specs=[pl.BlockSpec((blk, FP), lambda i: (i, 0)),
                   pl.BlockSpec((blk, FP), lambda i: (i, 0))],
        out_shape=[jax.ShapeDtypeStruct((M, FP), jnp.float32),
                   jax.ShapeDtypeStruct((M, FP), jnp.float32)],
        interpret=False,
    )(x, CF, SF)


def _xinv_kernel(ar_ref, ai_ref, br_ref, bi_ref, icc_ref, ics_ref, o_ref):
    ar, ai = ar_ref[...], ai_ref[...]
    br, bi = br_ref[...], bi_ref[...]
    sr = ar * br + ai * bi
    si = ar * bi - ai * br
    o_ref[...] = _dot(sr, icc_ref[...]) + _dot(si, ics_ref[...])


def _xcorr_inv(Ar, Ai, Br, Bi, ICc, ICs, blk):
    M, FP = Ar.shape
    L = ICc.shape[1]
    return pl.pallas_call(
        _xinv_kernel,
        grid=(M // blk,),
        in_specs=[pl.BlockSpec((blk, FP), lambda i: (i, 0)),
                  pl.BlockSpec((blk, FP), lambda i: (i, 0)),
                  pl.BlockSpec((blk, FP), lambda i: (i, 0)),
                  pl.BlockSpec((blk, FP), lambda i: (i, 0)),
                  pl.BlockSpec((FP, L), lambda i: (0, 0)),
                  pl.BlockSpec((FP, L), lambda i: (0, 0))],
        out_specs=pl.BlockSpec((blk, L), lambda i: (i, 0)),
        out_shape=jax.ShapeDtypeStruct((M, L), jnp.float32),
        interpret=False,
    )(Ar, Ai, Br, Bi, ICc, ICs)


def _topk_c_kernel(r_ref, c_ref, *, L, k):
    vals = r_ref[...]
    iota = jax.lax.broadcasted_iota(jnp.int32, vals.shape, 1)
    ws, ds = [], []
    for _ in range(k):
        m = jnp.max(vals, axis=1, keepdims=True)
        hit = vals == m
        idx = jnp.min(jnp.where(hit, iota, L), axis=1, keepdims=True)
        sel = iota == idx
        ws.append(m)
        ds.append(idx)
        vals = jnp.where(sel, -jnp.inf, vals)
    w = jnp.concatenate(ws, axis=1)
    p = jax.nn.softmax(w, axis=1)
    acc = jnp.zeros(r_ref.shape, jnp.float32)
    for i in range(k):
        acc = acc + jnp.where(iota == ds[i], p[:, i:i + 1], 0.0)
    c_ref[...] = acc


def _topk_c(R, k, blk):
    M, L = R.shape
    import functools
    return pl.pallas_call(
        functools.partial(_topk_c_kernel, L=L, k=k),
        grid=(M // blk,),
        in_specs=[pl.BlockSpec((blk, L), lambda i: (i, 0))],
        out_specs=pl.BlockSpec((blk, L), lambda i: (i, 0)),
        out_shape=jax.ShapeDtypeStruct((M, L), jnp.float32),
        interpret=False,
    )(R)


def kernel(q, k, v, Wq, bq, Wk, bk, Wv, bv, Wo, bo):
    B, L, D = q.shape
    H = _H
    depth = D // H
    lanes = B * H * depth
    kk = int(2 * math.log(L))

    mm_blk = min(512, B * L)
    lane_blk = min(256, lanes)

    qp = _matmul_bias(q.reshape(B * L, D), Wq, bq, mm_blk)
    kp = _matmul_bias(k.reshape(B * L, D), Wk, bk, mm_blk)

    def to_lanes(x):
        return x.reshape(B, L, H, depth).transpose(0, 2, 3, 1).reshape(lanes, L)

    qt = to_lanes(qp)
    kt = to_lanes(kp)

    CF, SF, ICc, ICs = _dft_mats(L)
    Qr, Qi = _fwd_fft(qt, CF, SF, lane_blk)
    Kr, Ki = _fwd_fft(kt, CF, SF, lane_blk)
    R = _xcorr_inv(Qr, Qi, Kr, Ki, ICc, ICs, lane_blk)
    c = _topk_c(R, kk, lane_blk)
    Cr, Ci = _fwd_fft(c, CF, SF, lane_blk)
    agg = _xcorr_inv(Qr, Qi, Cr, Ci, ICc, ICs, lane_blk)

    da = agg.reshape(B, H, depth, L).transpose(0, 3, 1, 2).reshape(B * L, D)
    out = _matmul_bias(da, Wo, bo, mm_blk)
    return out.reshape(B, L, D)


# speedup vs baseline: 10.4297x; 1.4990x over previous
import functools
import math

import numpy as np
import jax
import jax.numpy as jnp
from jax.experimental import pallas as pl

_H = 16


def _bdot(a, b):
    return jax.lax.dot_general(
        a.astype(jnp.bfloat16), b.astype(jnp.bfloat16),
        (((1,), (0,)), ((), ())), preferred_element_type=jnp.float32)


def _split_bf16(x):
    hi = x.astype(jnp.bfloat16)
    lo = (x - hi.astype(jnp.float32)).astype(jnp.bfloat16)
    return hi, lo


def _dot3(a, bh, bl):
    ah, al = _split_bf16(a)
    return _bdot(ah, bh) + (_bdot(ah, bl) + _bdot(al, bh))


def _dft_mats(L):
    F = L // 2 + 1
    FP = ((F + 127) // 128) * 128
    t = np.arange(L, dtype=np.int64)[:, None]
    f = np.arange(F, dtype=np.int64)[None, :]
    ang = 2.0 * np.pi * ((t * f) % L).astype(np.float64) / L
    CF = np.zeros((L, FP), np.float32)
    SF = np.zeros((L, FP), np.float32)
    CF[:, :F] = np.cos(ang)
    SF[:, :F] = np.sin(ang)
    alpha = np.full((F,), 2.0, np.float64)
    alpha[0] = 1.0
    if L % 2 == 0:
        alpha[F - 1] = 1.0
    ICc = np.zeros((FP, L), np.float32)
    ICs = np.zeros((FP, L), np.float32)
    ICc[:F, :] = (alpha[:, None] / L) * np.cos(ang.T)
    ICs[:F, :] = -(alpha[:, None] / L) * np.sin(ang.T)

    def split(m):
        hi = m.astype(np.dtype(jnp.bfloat16))
        lo = (m - hi.astype(np.float32)).astype(np.dtype(jnp.bfloat16))
        return jnp.asarray(hi), jnp.asarray(lo)

    return split(CF), split(SF), split(ICc), split(ICs)


def _mm_bias_kernel(x_ref, w_ref, b_ref, o_ref):
    o_ref[...] = _bdot(x_ref[...], w_ref[...]) + b_ref[...]


def _matmul_bias(x, W, b, blk):
    M, K = x.shape
    N = W.shape[1]
    return pl.pallas_call(
        _mm_bias_kernel,
        grid=(M // blk,),
        in_specs=[pl.BlockSpec((blk, K), lambda i: (i, 0)),
                  pl.BlockSpec((K, N), lambda i: (0, 0)),
                  pl.BlockSpec((1, N), lambda i: (0, 0))],
        out_specs=pl.BlockSpec((blk, N), lambda i: (i, 0)),
        out_shape=jax.ShapeDtypeStruct((M, N), jnp.float32),
        interpret=False,
    )(x, W, b.reshape(1, N))


def _fwd_kernel(x_ref, cfh_ref, cfl_ref, sfh_ref, sfl_ref, xr_ref, xi_ref):
    xh, xl = _split_bf16(x_ref[...])
    cfh, cfl = cfh_ref[...], cfl_ref[...]
    sfh, sfl = sfh_ref[...], sfl_ref[...]
    xr_ref[...] = _bdot(xh, cfh) + (_bdot(xh, cfl) + _bdot(xl, cfh))
    xi_ref[...] = _bdot(xh, sfh) + (_bdot(xh, sfl) + _bdot(xl, sfh))


def _fwd_fft(x, CFp, SFp, blk):
    M, L = x.shape
    FP = CFp[0].shape[1]
    mat = lambda: pl.BlockSpec((L, FP), lambda i: (0, 0))
    return pl.pallas_call(
        _fwd_kernel,
        grid=(M // blk,),
        in_specs=[pl.BlockSpec((blk, L), lambda i: (i, 0)),
                  mat(), mat(), mat(), mat()],
        out_specs=[pl.BlockSpec((blk, FP), lambda i: (i, 0)),
                   pl.BlockSpec((blk, FP), lambda i: (i, 0))],
        out_shape=[jax.ShapeDtypeStruct((M, FP), jnp.float32),
                   jax.ShapeDtypeStruct((M, FP), jnp.float32)],
        interpret=False,
    )(x, CFp[0], CFp[1], SFp[0], SFp[1])


def _xinv_kernel(ar_ref, ai_ref, br_ref, bi_ref, icch_ref, iccl_ref,
                 icsh_ref, icsl_ref, o_ref):
    ar, ai = ar_ref[...], ai_ref[...]
    br, bi = br_ref[...], bi_ref[...]
    sr = ar * br + ai * bi
    si = ar * bi - ai * br
    o_ref[...] = (_dot3(sr, icch_ref[...], iccl_ref[...])
                  + _dot3(si, icsh_ref[...], icsl_ref[...]))


def _xcorr_inv(Ar, Ai, Br, Bi, ICcp, ICsp, blk):
    M, FP = Ar.shape
    L = ICcp[0].shape[1]
    row = lambda: pl.BlockSpec((blk, FP), lambda i: (i, 0))
    mat = lambda: pl.BlockSpec((FP, L), lambda i: (0, 0))
    return pl.pallas_call(
        _xinv_kernel,
        grid=(M // blk,),
        in_specs=[row(), row(), row(), row(), mat(), mat(), mat(), mat()],
        out_specs=pl.BlockSpec((blk, L), lambda i: (i, 0)),
        out_shape=jax.ShapeDtypeStruct((M, L), jnp.float32),
        interpret=False,
    )(Ar, Ai, Br, Bi, ICcp[0], ICcp[1], ICsp[0], ICsp[1])


def _topk_c_kernel(r_ref, c_ref, *, L, k):
    vals = r_ref[...]
    iota = jax.lax.broadcasted_iota(jnp.int32, vals.shape, 1)
    ws, ds = [], []
    for _ in range(k):
        m = jnp.max(vals, axis=1, keepdims=True)
        hit = vals == m
        idx = jnp.min(jnp.where(hit, iota, L), axis=1, keepdims=True)
        sel = iota == idx
        ws.append(m)
        ds.append(idx)
        vals = jnp.where(sel, -jnp.inf, vals)
    w = jnp.concatenate(ws, axis=1)
    p = jax.nn.softmax(w, axis=1)
    acc = jnp.zeros(r_ref.shape, jnp.float32)
    for i in range(k):
        acc = acc + jnp.where(iota == ds[i], p[:, i:i + 1], 0.0)
    c_ref[...] = acc


def _topk_c(R, k, blk):
    M, L = R.shape
    return pl.pallas_call(
        functools.partial(_topk_c_kernel, L=L, k=k),
        grid=(M // blk,),
        in_specs=[pl.BlockSpec((blk, L), lambda i: (i, 0))],
        out_specs=pl.BlockSpec((blk, L), lambda i: (i, 0)),
        out_shape=jax.ShapeDtypeStruct((M, L), jnp.float32),
        interpret=False,
    )(R)


def kernel(q, k, v, Wq, bq, Wk, bk, Wv, bv, Wo, bo):
    B, L, D = q.shape
    H = _H
    depth = D // H
    lanes = B * H * depth
    kk = int(2 * math.log(L))

    mm_blk = min(512, B * L)
    lane_blk = min(256, lanes)

    qp = _matmul_bias(q.reshape(B * L, D), Wq, bq, mm_blk)
    kp = _matmul_bias(k.reshape(B * L, D), Wk, bk, mm_blk)

    def to_lanes(x):
        return x.reshape(B, L, H, depth).transpose(0, 2, 3, 1).reshape(lanes, L)

    qt = to_lanes(qp)
    kt = to_lanes(kp)

    CFp, SFp, ICcp, ICsp = _dft_mats(L)
    Qr, Qi = _fwd_fft(qt, CFp, SFp, lane_blk)
    Kr, Ki = _fwd_fft(kt, CFp, SFp, lane_blk)
    R = _xcorr_inv(Qr, Qi, Kr, Ki, ICcp, ICsp, lane_blk)
    c = _topk_c(R, kk, lane_blk)
    Cr, Ci = _fwd_fft(c, CFp, SFp, lane_blk)
    agg = _xcorr_inv(Qr, Qi, Cr, Ci, ICcp, ICsp, lane_blk)

    da = agg.reshape(B, H, depth, L).transpose(0, 3, 1, 2).reshape(B * L, D)
    out = _matmul_bias(da, Wo, bo, mm_blk)
    return out.reshape(B, L, D)


# speedup vs baseline: 13.0522x; 1.2514x over previous
import functools
import math

import numpy as np
import jax
import jax.numpy as jnp
from jax.experimental import pallas as pl

_H = 16


def _bdot(a, b):
    return jax.lax.dot_general(
        a.astype(jnp.bfloat16), b.astype(jnp.bfloat16),
        (((1,), (0,)), ((), ())), preferred_element_type=jnp.float32)


def _split_bf16(x):
    hi = x.astype(jnp.bfloat16)
    lo = (x - hi.astype(jnp.float32)).astype(jnp.bfloat16)
    return hi, lo


def _dot3(a, bh, bl):
    ah, al = _split_bf16(a)
    return _bdot(ah, bh) + (_bdot(ah, bl) + _bdot(al, bh))


def _dft_mats(L):
    F = L // 2 + 1
    FP = ((F + 127) // 128) * 128
    t = np.arange(L, dtype=np.int64)[:, None]
    f = np.arange(F, dtype=np.int64)[None, :]
    ang = 2.0 * np.pi * ((t * f) % L).astype(np.float64) / L
    CF = np.zeros((L, FP), np.float32)
    SF = np.zeros((L, FP), np.float32)
    CF[:, :F] = np.cos(ang)
    SF[:, :F] = np.sin(ang)
    alpha = np.full((F,), 2.0, np.float64)
    alpha[0] = 1.0
    if L % 2 == 0:
        alpha[F - 1] = 1.0
    ICc = np.zeros((FP, L), np.float32)
    ICs = np.zeros((FP, L), np.float32)
    ICc[:F, :] = (alpha[:, None] / L) * np.cos(ang.T)
    ICs[:F, :] = -(alpha[:, None] / L) * np.sin(ang.T)

    def split(m):
        hi = m.astype(np.dtype(jnp.bfloat16))
        lo = (m - hi.astype(np.float32)).astype(np.dtype(jnp.bfloat16))
        return jnp.asarray(hi), jnp.asarray(lo)

    return split(CF), split(SF), split(ICc), split(ICs)


def _proj_T_kernel(x_ref, w_ref, b_ref, o_ref):
    y = _bdot(x_ref[...], w_ref[...]) + b_ref[...]
    o_ref[...] = y.T


def _proj_to_lanes(x, W, b, B, L, tblk):
    M, K = x.shape
    N = W.shape[1]
    TB = L // tblk
    return pl.pallas_call(
        _proj_T_kernel,
        grid=(M // tblk,),
        in_specs=[pl.BlockSpec((tblk, K), lambda i: (i, 0)),
                  pl.BlockSpec((K, N), lambda i: (0, 0)),
                  pl.BlockSpec((1, N), lambda i: (0, 0))],
        out_specs=pl.BlockSpec((N, tblk), lambda i: (i // TB, i % TB)),
        out_shape=jax.ShapeDtypeStruct((B * N, L), jnp.float32),
        interpret=False,
    )(x, W, b.reshape(1, N))


def _mm_from_lanes_kernel(x_ref, w_ref, b_ref, o_ref):
    o_ref[...] = _bdot(x_ref[...].T, w_ref[...]) + b_ref[...]


def _mm_from_lanes(xt, W, b, B, L, tblk):
    D = W.shape[0]
    N = W.shape[1]
    TB = L // tblk
    return pl.pallas_call(
        _mm_from_lanes_kernel,
        grid=(B * TB,),
        in_specs=[pl.BlockSpec((D, tblk), lambda i: (i // TB, i % TB)),
                  pl.BlockSpec((D, N), lambda i: (0, 0)),
                  pl.BlockSpec((1, N), lambda i: (0, 0))],
        out_specs=pl.BlockSpec((tblk, N), lambda i: (i, 0)),
        out_shape=jax.ShapeDtypeStruct((B * L, N), jnp.float32),
        interpret=False,
    )(xt, W, b.reshape(1, N))


def _fwd_kernel(x_ref, cfh_ref, cfl_ref, sfh_ref, sfl_ref, xr_ref, xi_ref):
    xh, xl = _split_bf16(x_ref[...])
    cfh, cfl = cfh_ref[...], cfl_ref[...]
    sfh, sfl = sfh_ref[...], sfl_ref[...]
    xr_ref[...] = _bdot(xh, cfh) + (_bdot(xh, cfl) + _bdot(xl, cfh))
    xi_ref[...] = _bdot(xh, sfh) + (_bdot(xh, sfl) + _bdot(xl, sfh))


def _fwd_fft(x, CFp, SFp, blk):
    M, L = x.shape
    FP = CFp[0].shape[1]
    mat = lambda: pl.BlockSpec((L, FP), lambda i: (0, 0))
    return pl.pallas_call(
        _fwd_kernel,
        grid=(M // blk,),
        in_specs=[pl.BlockSpec((blk, L), lambda i: (i, 0)),
                  mat(), mat(), mat(), mat()],
        out_specs=[pl.BlockSpec((blk, FP), lambda i: (i, 0)),
                   pl.BlockSpec((blk, FP), lambda i: (i, 0))],
        out_shape=[jax.ShapeDtypeStruct((M, FP), jnp.float32),
                   jax.ShapeDtypeStruct((M, FP), jnp.float32)],
        interpret=False,
    )(x, CFp[0], CFp[1], SFp[0], SFp[1])


def _xinv_kernel(ar_ref, ai_ref, br_ref, bi_ref, icch_ref, iccl_ref,
                 icsh_ref, icsl_ref, o_ref):
    ar, ai = ar_ref[...], ai_ref[...]
    br, bi = br_ref[...], bi_ref[...]
    sr = ar * br + ai * bi
    si = ar * bi - ai * br
    o_ref[...] = (_dot3(sr, icch_ref[...], iccl_ref[...])
                  + _dot3(si, icsh_ref[...], icsl_ref[...]))


def _xcorr_inv(Ar, Ai, Br, Bi, ICcp, ICsp, blk):
    M, FP = Ar.shape
    L = ICcp[0].shape[1]
    row = lambda: pl.BlockSpec((blk, FP), lambda i: (i, 0))
    mat = lambda: pl.BlockSpec((FP, L), lambda i: (0, 0))
    return pl.pallas_call(
        _xinv_kernel,
        grid=(M // blk,),
        in_specs=[row(), row(), row(), row(), mat(), mat(), mat(), mat()],
        out_specs=pl.BlockSpec((blk, L), lambda i: (i, 0)),
        out_shape=jax.ShapeDtypeStruct((M, L), jnp.float32),
        interpret=False,
    )(Ar, Ai, Br, Bi, ICcp[0], ICcp[1], ICsp[0], ICsp[1])


def _topk_c_kernel(r_ref, c_ref, *, L, k):
    vals = r_ref[...]
    iota = jax.lax.broadcasted_iota(jnp.int32, vals.shape, 1)
    ws, ds = [], []
    for _ in range(k):
        m = jnp.max(vals, axis=1, keepdims=True)
        hit = vals == m
        idx = jnp.min(jnp.where(hit, iota, L), axis=1, keepdims=True)
        sel = iota == idx
        ws.append(m)
        ds.append(idx)
        vals = jnp.where(sel, -jnp.inf, vals)
    w = jnp.concatenate(ws, axis=1)
    p = jax.nn.softmax(w, axis=1)
    acc = jnp.zeros(r_ref.shape, jnp.float32)
    for i in range(k):
        acc = acc + jnp.where(iota == ds[i], p[:, i:i + 1], 0.0)
    c_ref[...] = acc


def _topk_c(R, k, blk):
    M, L = R.shape
    return pl.pallas_call(
        functools.partial(_topk_c_kernel, L=L, k=k),
        grid=(M // blk,),
        in_specs=[pl.BlockSpec((blk, L), lambda i: (i, 0))],
        out_specs=pl.BlockSpec((blk, L), lambda i: (i, 0)),
        out_shape=jax.ShapeDtypeStruct((M, L), jnp.float32),
        interpret=False,
    )(R)


def kernel(q, k, v, Wq, bq, Wk, bk, Wv, bv, Wo, bo):
    B, L, D = q.shape
    H = _H
    depth = D // H
    lanes = B * H * depth
    kk = int(2 * math.log(L))

    mm_blk = min(512, L)
    lane_blk = min(256, lanes)

    qt = _proj_to_lanes(q.reshape(B * L, D), Wq, bq, B, L, mm_blk)
    kt = _proj_to_lanes(k.reshape(B * L, D), Wk, bk, B, L, mm_blk)

    CFp, SFp, ICcp, ICsp = _dft_mats(L)
    Qr, Qi = _fwd_fft(qt, CFp, SFp, lane_blk)
    Kr, Ki = _fwd_fft(kt, CFp, SFp, lane_blk)
    R = _xcorr_inv(Qr, Qi, Kr, Ki, ICcp, ICsp, lane_blk)
    c = _topk_c(R, kk, lane_blk)
    Cr, Ci = _fwd_fft(c, CFp, SFp, lane_blk)
    agg = _xcorr_inv(Qr, Qi, Cr, Ci, ICcp, ICsp, lane_blk)

    out = _mm_from_lanes(agg, Wo, bo, B, L, mm_blk)
    return out.reshape(B, L, D)
